# baseline (speedup 1.0000x reference)
import functools

import jax
import jax.numpy as jnp
from jax import lax
from jax.experimental import pallas as pl
from jax.experimental.pallas import tpu as pltpu

E = 8
H = 2048
F = 1408
TOPK = 2


def _router_body(x_ref, rwt_ref, comb_ref):
    x = x_ref[...]
    logits = jnp.dot(x, rwt_ref[...], preferred_element_type=jnp.float32)
    m = jnp.max(logits, axis=-1, keepdims=True)
    ex = jnp.exp(logits - m)
    aff = ex / jnp.sum(ex, axis=-1, keepdims=True)
    idx = lax.broadcasted_iota(jnp.int32, aff.shape, 1)
    m1 = jnp.max(aff, axis=-1, keepdims=True)
    i1 = jnp.min(jnp.where(aff == m1, idx, E), axis=-1, keepdims=True)
    aff2 = jnp.where(idx == i1, -1.0, aff)
    m2 = jnp.max(aff2, axis=-1, keepdims=True)
    i2 = jnp.min(jnp.where(aff2 == m2, idx, E), axis=-1, keepdims=True)
    s = m1 + m2
    w1 = m1 / s
    w2 = m2 / s
    comb_ref[...] = jnp.where(idx == i1, w1, 0.0) + jnp.where(idx == i2, w2, 0.0)


def _moe_body(x_ref, wg_ref, wu_ref, wd_ref, c_ref, o_ref, acc_ref):
    e = pl.program_id(1)
    f = pl.program_id(2)

    @pl.when((e == 0) & (f == 0))
    def _():
        acc_ref[...] = jnp.zeros_like(acc_ref)

    xb = x_ref[...]
    g = jnp.dot(xb, wg_ref[0], preferred_element_type=jnp.float32)
    u = jnp.dot(xb, wu_ref[0], preferred_element_type=jnp.float32)
    a = (g * jax.nn.sigmoid(g)) * u
    p = jnp.dot(a, wd_ref[0], preferred_element_type=jnp.float32)
    onehot = (lax.broadcasted_iota(jnp.int32, (E, 1), 0) == e).astype(jnp.float32)
    ce = jnp.dot(c_ref[...], onehot, preferred_element_type=jnp.float32)
    acc_ref[...] += p * ce

    @pl.when((e == E - 1) & (f == pl.num_programs(2) - 1))
    def _():
        o_ref[...] = acc_ref[...]


def kernel(hidden_states, router_w, w_gate, w_up, w_down):
    b, s, h = hidden_states.shape
    T = b * s
    x = hidden_states.reshape(T, h)

    combine = pl.pallas_call(
        _router_body,
        out_shape=jax.ShapeDtypeStruct((T, E), jnp.float32),
    )(x, router_w.T)

    TM = 1024
    FT = 128
    grid = (T // TM, E, F // FT)
    out = pl.pallas_call(
        _moe_body,
        grid=grid,
        in_specs=[
            pl.BlockSpec((TM, H), lambda t, e, f: (t, 0)),
            pl.BlockSpec((1, H, FT), lambda t, e, f: (e, 0, f)),
            pl.BlockSpec((1, H, FT), lambda t, e, f: (e, 0, f)),
            pl.BlockSpec((1, FT, H), lambda t, e, f: (e, f, 0)),
            pl.BlockSpec((TM, E), lambda t, e, f: (t, 0)),
        ],
        out_specs=pl.BlockSpec((TM, H), lambda t, e, f: (t, 0)),
        out_shape=jax.ShapeDtypeStruct((T, H), jnp.float32),
        scratch_shapes=[pltpu.VMEM((TM, H), jnp.float32)],
        compiler_params=pltpu.CompilerParams(
            dimension_semantics=("arbitrary", "arbitrary", "arbitrary"),
        ),
    )(x, w_gate, w_up, w_down, combine)

    return out.reshape(b, s, h)


# speedup vs baseline: 1.5742x; 1.5742x over previous
import functools

import jax
import jax.numpy as jnp
from jax import lax
from jax.experimental import pallas as pl
from jax.experimental.pallas import tpu as pltpu

E = 8
H = 2048
F = 1408
TOPK = 2


def _router_body(x_ref, rwt_ref, comb_ref):
    x = x_ref[...]
    logits = jnp.dot(x, rwt_ref[...], preferred_element_type=jnp.float32)
    m = jnp.max(logits, axis=-1, keepdims=True)
    ex = jnp.exp(logits - m)
    aff = ex / jnp.sum(ex, axis=-1, keepdims=True)
    idx = lax.broadcasted_iota(jnp.int32, aff.shape, 1)
    m1 = jnp.max(aff, axis=-1, keepdims=True)
    i1 = jnp.min(jnp.where(aff == m1, idx, E), axis=-1, keepdims=True)
    aff2 = jnp.where(idx == i1, -1.0, aff)
    m2 = jnp.max(aff2, axis=-1, keepdims=True)
    i2 = jnp.min(jnp.where(aff2 == m2, idx, E), axis=-1, keepdims=True)
    s = m1 + m2
    w1 = m1 / s
    w2 = m2 / s
    comb_ref[...] = jnp.where(idx == i1, w1, 0.0) + jnp.where(idx == i2, w2, 0.0)


def _moe_body(x_ref, wg_ref, wu_ref, wd_ref, c_ref, o_ref, acc_ref):
    e = pl.program_id(1)

    @pl.when(e == 0)
    def _():
        acc_ref[...] = jnp.zeros_like(acc_ref)

    xb = x_ref[...]
    g = jnp.dot(xb, wg_ref[0], preferred_element_type=jnp.float32)
    u = jnp.dot(xb, wu_ref[0], preferred_element_type=jnp.float32)
    a = ((g * jax.nn.sigmoid(g)) * u).astype(jnp.bfloat16)
    p = jnp.dot(a, wd_ref[0], preferred_element_type=jnp.float32)
    onehot = (lax.broadcasted_iota(jnp.int32, (E, 1), 0) == e).astype(jnp.float32)
    ce = jnp.dot(c_ref[...], onehot, preferred_element_type=jnp.float32)
    acc_ref[...] += p * ce

    @pl.when(e == E - 1)
    def _():
        o_ref[...] = acc_ref[...]


def kernel(hidden_states, router_w, w_gate, w_up, w_down):
    b, s, h = hidden_states.shape
    T = b * s
    x = hidden_states.reshape(T, h)

    combine = pl.pallas_call(
        _router_body,
        out_shape=jax.ShapeDtypeStruct((T, E), jnp.float32),
    )(x, router_w.T)

    xb = x.astype(jnp.bfloat16)
    wg = w_gate.astype(jnp.bfloat16)
    wu = w_up.astype(jnp.bfloat16)
    wd = w_down.astype(jnp.bfloat16)

    TM = 512
    grid = (T // TM, E)
    out = pl.pallas_call(
        _moe_body,
        grid=grid,
        in_specs=[
            pl.BlockSpec((TM, H), lambda t, e: (t, 0)),
            pl.BlockSpec((1, H, F), lambda t, e: (e, 0, 0)),
            pl.BlockSpec((1, H, F), lambda t, e: (e, 0, 0)),
            pl.BlockSpec((1, F, H), lambda t, e: (e, 0, 0)),
            pl.BlockSpec((TM, E), lambda t, e: (t, 0)),
        ],
        out_specs=pl.BlockSpec((TM, H), lambda t, e: (t, 0)),
        out_shape=jax.ShapeDtypeStruct((T, H), jnp.float32),
        scratch_shapes=[pltpu.VMEM((TM, H), jnp.float32)],
        compiler_params=pltpu.CompilerParams(
            dimension_semantics=("arbitrary", "arbitrary"),
        ),
    )(xb, wg, wu, wd, combine)

    return out.reshape(b, s, h)


# speedup vs baseline: 1.8318x; 1.1636x over previous
import functools

import jax
import jax.numpy as jnp
from jax import lax
from jax.experimental import pallas as pl
from jax.experimental.pallas import tpu as pltpu

E = 8
H = 2048
F = 1408
T = 2048
TM = 256
NJ = (2 * T) // TM + E
NS = NJ * TM


def _router_body(x_ref, rwt_ref, x16_ref, pos_ref, wexp_ref, toff_ref):
    x = x_ref[...]
    x16_ref[...] = x.astype(jnp.bfloat16)
    logits = jnp.dot(x, rwt_ref[...], preferred_element_type=jnp.float32)
    m = jnp.max(logits, axis=-1, keepdims=True)
    ex = jnp.exp(logits - m)
    aff = ex / jnp.sum(ex, axis=-1, keepdims=True)
    idx = lax.broadcasted_iota(jnp.int32, aff.shape, 1)
    m1 = jnp.max(aff, axis=-1, keepdims=True)
    i1 = jnp.min(jnp.where(aff == m1, idx, E), axis=-1, keepdims=True)
    aff2 = jnp.where(idx == i1, -1.0, aff)
    m2 = jnp.max(aff2, axis=-1, keepdims=True)
    i2 = jnp.min(jnp.where(aff2 == m2, idx, E), axis=-1, keepdims=True)
    s = m1 + m2
    wexp_ref[0] = jnp.broadcast_to(m1 / s, (T, 16))
    wexp_ref[1] = jnp.broadcast_to(m2 / s, (T, 16))

    oh1 = (idx == i1).astype(jnp.float32)
    oh2 = (idx == i2).astype(jnp.float32)
    tri = (lax.broadcasted_iota(jnp.int32, (128, 128), 0)
           >= lax.broadcasted_iota(jnp.int32, (128, 128), 1)).astype(jnp.float32)

    def _cumsum_tokens(oh):
        outs = []
        prefix = jnp.zeros((1, E), jnp.float32)
        for blk in range(T // 128):
            part = oh[blk * 128:(blk + 1) * 128, :]
            cw = jnp.dot(tri, part, preferred_element_type=jnp.float32) + prefix
            outs.append(cw)
            prefix = cw[127:128, :]
        return jnp.concatenate(outs, axis=0)

    c1 = _cumsum_tokens(oh1)
    c2 = _cumsum_tokens(oh2)
    n1 = c1[T - 1:T, :]
    counts = n1 + c2[T - 1:T, :]
    nt = jnp.ceil(counts / TM)
    lane = lax.broadcasted_iota(jnp.int32, (E, E), 0)
    lane2 = lax.broadcasted_iota(jnp.int32, (E, E), 1)
    strict_lt = (lane < lane2).astype(jnp.float32)
    toff = jnp.dot(nt, strict_lt, preferred_element_type=jnp.float32)
    off_pad = toff * TM
    pos1 = jnp.sum(oh1 * (off_pad + c1 - oh1), axis=-1, keepdims=True)
    pos2 = jnp.sum(oh2 * (off_pad + n1 + c2 - oh2), axis=-1, keepdims=True)
    pos_pair = jnp.concatenate([pos1, pos2], axis=1).astype(jnp.int32)
    pos_ref[...] = pos_pair.T
    toff_ref[...] = toff.astype(jnp.int32)


def _gemm_body(g_ref, xs_ref, wg_ref, wu_ref, wd_ref, ys_ref):
    xb = xs_ref[...]
    g = jnp.dot(xb, wg_ref[0], preferred_element_type=jnp.float32)
    u = jnp.dot(xb, wu_ref[0], preferred_element_type=jnp.float32)
    a = ((g * jax.nn.sigmoid(g)) * u).astype(jnp.bfloat16)
    ys_ref[...] = jnp.dot(a, wd_ref[0], preferred_element_type=jnp.float32)


def kernel(hidden_states, router_w, w_gate, w_up, w_down):
    b, s, h = hidden_states.shape
    x = hidden_states.reshape(T, h)

    x16, pos, wexp, toff = pl.pallas_call(
        _router_body,
        out_shape=[
            jax.ShapeDtypeStruct((T, H), jnp.bfloat16),
            jax.ShapeDtypeStruct((2, T), jnp.int32),
            jax.ShapeDtypeStruct((2, T, 16), jnp.float32),
            jax.ShapeDtypeStruct((1, E), jnp.int32),
        ],
    )(x, router_w.T)

    g_arr = jnp.clip(
        jnp.sum(jnp.arange(NJ, dtype=jnp.int32)[:, None] >= toff[0][None, :], axis=1) - 1,
        0, E - 1).astype(jnp.int32)

    xs16 = jnp.zeros((NS, H), jnp.bfloat16).at[pos[0]].set(x16).at[pos[1]].set(x16)

    wg16 = w_gate.astype(jnp.bfloat16)
    wu16 = w_up.astype(jnp.bfloat16)
    wd16 = w_down.astype(jnp.bfloat16)

    grid_spec = pltpu.PrefetchScalarGridSpec(
        num_scalar_prefetch=1,
        grid=(NJ,),
        in_specs=[
            pl.BlockSpec((TM, H), lambda j, g: (j, 0)),
            pl.BlockSpec((1, H, F), lambda j, g: (g[j], 0, 0)),
            pl.BlockSpec((1, H, F), lambda j, g: (g[j], 0, 0)),
            pl.BlockSpec((1, F, H), lambda j, g: (g[j], 0, 0)),
        ],
        out_specs=pl.BlockSpec((TM, H), lambda j, g: (j, 0)),
    )
    ys = pl.pallas_call(
        _gemm_body,
        grid_spec=grid_spec,
        out_shape=jax.ShapeDtypeStruct((NS, H), jnp.float32),
        compiler_params=pltpu.CompilerParams(
            dimension_semantics=("arbitrary",),
        ),
    )(g_arr, xs16, wg16, wu16, wd16)

    out = wexp[0, :, :1] * ys[pos[0]] + wexp[1, :, :1] * ys[pos[1]]
    return out.reshape(b, s, h)
